# baseline (speedup 1.0000x reference)
import functools

import jax
import jax.numpy as jnp
from jax import lax
from jax.experimental import pallas as pl
from jax.experimental.pallas import tpu as pltpu
from jax.experimental.pallas import tpu_sc as plsc

SEQ = 200
D = 64
BATCH = 4096
VEC = 16
NCORES = 2
NSUB = 16
NW = NCORES * NSUB
SEQ_PER_W = BATCH // NW
HALF = 100
NBUF = 4
LOOKAHEAD = 2


def _sc_body(inputs_hbm, base_hbm, pos_hbm, out_hbm,
             pos_v, idx0, idx1, idx2, idx3, r0, r1, r2, r3,
             g0, g1, g2, g3, s0, s1, s2, s3):
    idxb = (idx0, idx1, idx2, idx3)
    rows = (r0, r1, r2, r3)
    gsem = (g0, g1, g2, g3)
    ssem = (s0, s1, s2, s3)

    wid = lax.axis_index("c") * NSUB + lax.axis_index("s")
    base_seq = wid * SEQ_PER_W

    pltpu.sync_copy(pos_hbm, pos_v)

    def start_gather(k, b):
        pltpu.sync_copy(inputs_hbm.at[base_seq + k], idxb[b])
        pltpu.async_copy(base_hbm.at[idxb[b].at[0]],
                         rows[b].at[pl.ds(0, HALF)], gsem[b])
        pltpu.async_copy(base_hbm.at[idxb[b].at[1]],
                         rows[b].at[pl.ds(HALF, HALF)], gsem[b])

    def wait_gather(b):
        pltpu.make_async_copy(out_hbm.at[0], rows[b], gsem[b]).wait()

    def wait_scatter(b):
        pltpu.make_async_copy(rows[b], out_hbm.at[0], ssem[b]).wait()

    def add_pos(b):
        def body(i, c):
            for v in range(D // VEC):
                sl = pl.ds(v * VEC, VEC)
                plsc.addupdate(rows[b].at[i, sl], pos_v[i, sl])
            return c
        lax.fori_loop(0, SEQ, body, 0)

    for j in range(LOOKAHEAD):
        start_gather(j, j)

    def t_body(t, carry):
        for u in range(NBUF):
            k = t * NBUF + u
            bn = (u + LOOKAHEAD) % NBUF
            kn = k + LOOKAHEAD

            @pl.when(kn < SEQ_PER_W)
            def _():
                @pl.when(k >= LOOKAHEAD)
                def _():
                    wait_scatter(bn)
                start_gather(kn, bn)

            wait_gather(u)
            add_pos(u)
            pltpu.async_copy(rows[u], out_hbm.at[base_seq + k], ssem[u])
        return carry

    lax.fori_loop(0, SEQ_PER_W // NBUF, t_body, 0)

    for b in range(NBUF):
        wait_scatter(b)


@jax.jit
def _run(inputs3, base_table, pos_table):
    mesh = plsc.VectorSubcoreMesh(core_axis_name="c", subcore_axis_name="s")
    f = pl.kernel(
        _sc_body,
        out_type=jax.ShapeDtypeStruct((BATCH, SEQ, D), jnp.float32),
        mesh=mesh,
        compiler_params=pltpu.CompilerParams(use_tc_tiling_on_sc=False),
        scratch_types=[
            pltpu.VMEM((SEQ, D), jnp.float32),
            *[pltpu.VMEM((2, HALF), jnp.int32) for _ in range(NBUF)],
            *[pltpu.VMEM((SEQ, D), jnp.float32) for _ in range(NBUF)],
            *[pltpu.SemaphoreType.DMA for _ in range(2 * NBUF)],
        ],
    )
    return f(inputs3, base_table, pos_table)


def kernel(inputs, base_table, pos_table):
    inputs3 = inputs.astype(jnp.int32).reshape(BATCH, 2, HALF)
    return _run(inputs3, base_table, pos_table)


# speedup vs baseline: 1.0470x; 1.0470x over previous
import jax
import jax.numpy as jnp
from jax import lax
from jax.experimental import pallas as pl
from jax.experimental.pallas import tpu as pltpu
from jax.experimental.pallas import tpu_sc as plsc

SEQ = 200
D = 64
BATCH = 4096
VEC = 16
NCORES = 2
NSUB = 16
NW = NCORES * NSUB
BPW = BATCH // NW
NBUF = 4
LOOKAHEAD = 2
RUNROLL = 4


def _sc_body(inputs_hbm, base_hbm, pos_hbm, out_hbm,
             idx_v, pos_v, r0, r1, r2, r3,
             g0, g1, g2, g3, s0, s1, s2, s3):
    rows = (r0, r1, r2, r3)
    gsem = (g0, g1, g2, g3)
    ssem = (s0, s1, s2, s3)

    wid = lax.axis_index("c") * NSUB + lax.axis_index("s")
    b0 = wid * BPW

    pltpu.sync_copy(inputs_hbm.at[wid], idx_v)
    pltpu.sync_copy(pos_hbm, pos_v)

    def start_gather(l, b):
        pltpu.async_copy(base_hbm.at[idx_v.at[l]], rows[b], gsem[b])

    def wait_gather(b):
        pltpu.make_async_copy(out_hbm.at[pl.ds(0, BPW), 0], rows[b],
                              gsem[b]).wait()

    def wait_scatter(b):
        pltpu.make_async_copy(rows[b], out_hbm.at[pl.ds(0, BPW), 0],
                              ssem[b]).wait()

    def add_pos(l, b):
        pv = [pos_v[l, pl.ds(v * VEC, VEC)] for v in range(D // VEC)]

        def body(i, c):
            for r in range(RUNROLL):
                for v in range(D // VEC):
                    plsc.addupdate(
                        rows[b].at[i * RUNROLL + r, pl.ds(v * VEC, VEC)],
                        pv[v])
            return c
        lax.fori_loop(0, BPW // RUNROLL, body, 0)

    for j in range(LOOKAHEAD):
        start_gather(j, j)

    def t_body(t, carry):
        for u in range(NBUF):
            l = t * NBUF + u
            bn = (u + LOOKAHEAD) % NBUF
            ln = l + LOOKAHEAD

            @pl.when(ln < SEQ)
            def _():
                @pl.when(l >= LOOKAHEAD)
                def _():
                    wait_scatter(bn)
                start_gather(ln, bn)

            wait_gather(u)
            add_pos(l, u)
            pltpu.async_copy(rows[u], out_hbm.at[pl.ds(b0, BPW), l], ssem[u])
        return carry

    lax.fori_loop(0, SEQ // NBUF, t_body, 0)

    for b in range(NBUF):
        wait_scatter(b)


@jax.jit
def _run(inputs_t, base_table, pos_table):
    mesh = plsc.VectorSubcoreMesh(core_axis_name="c", subcore_axis_name="s")
    f = pl.kernel(
        _sc_body,
        out_type=jax.ShapeDtypeStruct((BATCH, SEQ, D), jnp.float32),
        mesh=mesh,
        compiler_params=pltpu.CompilerParams(use_tc_tiling_on_sc=False),
        scratch_types=[
            pltpu.VMEM((SEQ, BPW), jnp.int32),
            pltpu.VMEM((SEQ, D), jnp.float32),
            *[pltpu.VMEM((BPW, D), jnp.float32) for _ in range(NBUF)],
            *[pltpu.SemaphoreType.DMA for _ in range(2 * NBUF)],
        ],
    )
    return f(inputs_t, base_table, pos_table)


def kernel(inputs, base_table, pos_table):
    inputs_t = (inputs.astype(jnp.int32)
                .reshape(NW, BPW, SEQ)
                .transpose(0, 2, 1))
    return _run(inputs_t, base_table, pos_table)
